# baseline (speedup 1.0000x reference)
import functools
import jax
import jax.numpy as jnp
from jax import lax
from jax.experimental import pallas as pl
from jax.experimental.pallas import tpu as pltpu
from jax.experimental.pallas import tpu_sc as plsc

N = 10000
D = 128
E = 320000
OUT = 3
NC = 2
NS = 16
NW = NC * NS
L = 16
EPT = E // NW
CH = 80
NCH = EPT // CH
DEG_PAD = 10240
DSL = DEG_PAD // NS
SPAD = 10240
RPT = SPAD // NS
RSTG = 128
SB = 25
NSB = NCH // SB
RB = 1000


def _zero_f32(ref, n):
    z = jnp.zeros((L,), jnp.float32)

    def body(i, c):
        ref[pl.ds(i * L, L)] = z
        return c

    lax.fori_loop(0, n // L, body, 0)


def _zero_rows(ref, rows):
    z = jnp.zeros((L,), jnp.float32)

    def body(i, c):
        def inner(d, c2):
            ref[i, pl.ds(d * L, L)] = z
            return c2

        return lax.fori_loop(0, D // L, inner, c)

    lax.fori_loop(0, rows, body, 0)


_SC_MESH = plsc.VectorSubcoreMesh(core_axis_name="c", subcore_axis_name="s")


@functools.partial(
    pl.kernel,
    mesh=_SC_MESH,
    out_type=jax.ShapeDtypeStruct((NC, DEG_PAD), jnp.float32),
    scratch_types=[
        pltpu.VMEM((NCH, 1, CH), jnp.int32),
        pltpu.VMEM((EPT,), jnp.float32),
        pltpu.VMEM((DSL,), jnp.float32),
        pltpu.VMEM_SHARED((DEG_PAD,), jnp.float32),
    ],
)
def _deg_call(col3_hbm, ew_hbm, out_hbm, cidx_v, ew_v, stage_v, deg_sh):
    cid = lax.axis_index("c")
    sid = lax.axis_index("s")
    wid = sid * NC + cid
    _zero_f32(stage_v, DSL)
    pltpu.sync_copy(stage_v, deg_sh.at[pl.ds(sid * DSL, DSL)])
    pltpu.sync_copy(col3_hbm.at[pl.ds(wid * NCH, NCH)], cidx_v)
    pltpu.sync_copy(ew_hbm.at[pl.ds(wid * EPT, EPT)], ew_v)
    plsc.subcore_barrier()

    def chunk(c, carry):
        pltpu.sync_copy(ew_v.at[pl.ds(c * CH, CH)],
                        deg_sh.at[cidx_v.at[c, 0]], add=True)
        return carry

    lax.fori_loop(0, NCH, chunk, 0)
    plsc.subcore_barrier()
    pltpu.sync_copy(deg_sh.at[pl.ds(sid * DSL, DSL)], stage_v)
    pltpu.sync_copy(stage_v, out_hbm.at[cid, pl.ds(sid * DSL, DSL)])


def _bcast_lane(v16, j):
    return lax.gather(
        v16, jnp.full((L, 1), j, jnp.int32),
        lax.GatherDimensionNumbers(
            offset_dims=(), collapsed_slice_dims=(0,), start_index_map=(0,)),
        slice_sizes=(1,),
        mode=lax.GatherScatterMode.PROMISE_IN_BOUNDS)


@functools.partial(
    pl.kernel,
    mesh=_SC_MESH,
    out_type=jax.ShapeDtypeStruct((NC, SPAD, D), jnp.float32),
    scratch_types=[
        pltpu.VMEM((SB, 1, CH), jnp.int32),
        pltpu.VMEM((SB, 1, CH), jnp.int32),
        pltpu.VMEM((SB * CH,), jnp.float32),
        pltpu.VMEM((CH, D), jnp.float32),
        pltpu.VMEM((CH, D), jnp.float32),
        pltpu.VMEM_SHARED((SPAD, D), jnp.float32),
        pltpu.SemaphoreType.DMA,
        pltpu.SemaphoreType.DMA,
        pltpu.SemaphoreType.DMA,
        pltpu.SemaphoreType.DMA,
    ],
)
def _scatter_call(row3_hbm, col3_hbm, ew_hbm, y_hbm, out_hbm,
                  ridx_v, cidx_v, ew_v, rows_a, rows_b, s_sh,
                  sem_a, sem_b, ssem_a, ssem_b):
    cid = lax.axis_index("c")
    sid = lax.axis_index("s")
    wid = sid * NC + cid
    _zero_rows(rows_a, CH)
    for k in range(RPT // CH):
        pltpu.sync_copy(rows_a, s_sh.at[pl.ds(sid * RPT + k * CH, CH)])
    plsc.subcore_barrier()

    def gather(c, buf, sem):
        pltpu.async_copy(y_hbm.at[ridx_v.at[c, 0]], buf, sem)

    def wait_g(c, buf, sem):
        pltpu.make_async_copy(y_hbm.at[ridx_v.at[c, 0]], buf, sem).wait()

    def scale(c, buf):
        def group(g, c2):
            ew16 = ew_v[pl.ds(c * CH + g * L, L)]
            for j in range(L):
                w16 = _bcast_lane(ew16, j)
                e = g * L + j
                for d in range(D // L):
                    sl = pl.ds(d * L, L)
                    buf[e, sl] = buf[e, sl] * w16
            return c2

        lax.fori_loop(0, CH // L, group, 0)

    def issue_sc(c, buf, ssem):
        pltpu.async_copy(buf, s_sh.at[cidx_v.at[c, 0]], ssem, add=True)

    def wait_sc(c, buf, ssem):
        pltpu.make_async_copy(buf, s_sh.at[cidx_v.at[c, 0]], ssem).wait()

    def superblock(s, carry):
        cb = wid * NCH + s * SB
        pltpu.sync_copy(row3_hbm.at[pl.ds(cb, SB)], ridx_v)
        pltpu.sync_copy(col3_hbm.at[pl.ds(cb, SB)], cidx_v)
        pltpu.sync_copy(ew_hbm.at[pl.ds(wid * EPT + s * SB * CH, SB * CH)],
                        ew_v)
        gather(0, rows_a, sem_a)
        gather(1, rows_b, sem_b)

        def pair(i, c2):
            c0 = 2 * i
            wait_g(c0, rows_a, sem_a)
            scale(c0, rows_a)
            issue_sc(c0, rows_a, ssem_a)
            wait_g(c0 + 1, rows_b, sem_b)
            scale(c0 + 1, rows_b)
            issue_sc(c0 + 1, rows_b, ssem_b)
            wait_sc(c0, rows_a, ssem_a)
            gather(c0 + 2, rows_a, sem_a)
            wait_sc(c0 + 1, rows_b, ssem_b)
            gather(c0 + 3, rows_b, sem_b)
            return c2

        lax.fori_loop(0, (SB - 3) // 2, pair, 0)
        c0 = SB - 3
        wait_g(c0, rows_a, sem_a)
        scale(c0, rows_a)
        issue_sc(c0, rows_a, ssem_a)
        wait_g(c0 + 1, rows_b, sem_b)
        scale(c0 + 1, rows_b)
        issue_sc(c0 + 1, rows_b, ssem_b)
        wait_sc(c0, rows_a, ssem_a)
        gather(c0 + 2, rows_a, sem_a)
        wait_g(c0 + 2, rows_a, sem_a)
        scale(c0 + 2, rows_a)
        issue_sc(c0 + 2, rows_a, ssem_a)
        wait_sc(c0 + 1, rows_b, ssem_b)
        wait_sc(c0 + 2, rows_a, ssem_a)
        return carry

    lax.fori_loop(0, NSB, superblock, 0)
    plsc.subcore_barrier()
    for k in range(RPT // CH):
        r0 = sid * RPT + k * CH
        pltpu.sync_copy(s_sh.at[pl.ds(r0, CH)], rows_a)
        pltpu.sync_copy(rows_a, out_hbm.at[cid, pl.ds(r0, CH)])


def _y_body(nf_ref, pw_ref, pb_ref, emb_ref, degp_ref, y_ref):
    nf = nf_ref[...]
    x = (nf[:, 0:1] * pw_ref[0:1, :]
         + nf[:, 1:2] * pw_ref[1:2, :]
         + nf[:, 2:3] * pw_ref[2:3, :]
         + pb_ref[...] + emb_ref[...])
    deg = degp_ref[0] + degp_ref[1] + 1.0
    dis = lax.rsqrt(deg)
    y_ref[...] = dis * x


def _dense_body(nf_ref, pw_ref, pb_ref, emb_ref, degp_ref, sp_ref,
                wz_ref, bz_ref, wh_ref, bh_ref,
                lzw_ref, lzb_ref, lhw_ref, lhb_ref,
                w1_ref, b1_ref, w2_ref, b2_ref, out_ref):
    nf = nf_ref[...]
    x = (nf[:, 0:1] * pw_ref[0:1, :]
         + nf[:, 1:2] * pw_ref[1:2, :]
         + nf[:, 2:3] * pw_ref[2:3, :]
         + pb_ref[...] + emb_ref[...])
    deg = degp_ref[0] + degp_ref[1] + 1.0
    dis = lax.rsqrt(deg)
    s = sp_ref[0] + sp_ref[1]
    m = dis * s + (dis * dis) * x
    cz = jnp.dot(m, wz_ref[...], preferred_element_type=jnp.float32) + bz_ref[...]
    z = jax.nn.sigmoid(
        jnp.dot(cz, lzw_ref[...], preferred_element_type=jnp.float32) + lzb_ref[...])
    chh = jnp.dot(m, wh_ref[...], preferred_element_type=jnp.float32) + bh_ref[...]
    ht = jnp.tanh(
        jnp.dot(chh, lhw_ref[...], preferred_element_type=jnp.float32) + lhb_ref[...])
    hn = (1.0 - z) * ht
    hm = jnp.where(nf[:, 3:4] != -999.0, hn, 0.0)
    h1 = jax.nn.relu(
        jnp.dot(hm, w1_ref[...], preferred_element_type=jnp.float32) + b1_ref[...])
    out_ref[...] = (
        jnp.dot(h1, w2_ref[...], preferred_element_type=jnp.float32) + b2_ref[...])


def _row2d(v):
    return v.reshape(1, -1)


def kernel(dynamic_node_feats, node_ids, edge_index, edge_feats, proj_W,
           proj_b, node_embeddings, conv_z_W, conv_z_b, conv_r_W, conv_r_b,
           conv_h_W, conv_h_b, lin_z_W, lin_z_b, lin_r_W, lin_r_b, lin_h_W,
           lin_h_b, pred_W1, pred_b1, pred_W2, pred_b2):
    row = edge_index[0]
    col = edge_index[1]
    ew = edge_feats[:, 0]
    row3 = row.reshape(NW * NCH, 1, CH)
    col3 = col.reshape(NW * NCH, 1, CH)

    deg_p = _deg_call(col3, ew)
    degp3 = deg_p.reshape(NC, DEG_PAD, 1)

    nf = dynamic_node_feats
    grid = (N // RB,)

    def rowspec():
        return pl.BlockSpec((RB, D), lambda i: (i, 0))

    def smallspec(shape):
        return pl.BlockSpec(shape, lambda i: tuple(0 for _ in shape))

    y = pl.pallas_call(
        _y_body,
        grid=grid,
        in_specs=[
            pl.BlockSpec((RB, 4), lambda i: (i, 0)),
            smallspec((3, D)),
            smallspec((1, D)),
            rowspec(),
            pl.BlockSpec((NC, RB, 1), lambda i: (0, i, 0)),
        ],
        out_specs=rowspec(),
        out_shape=jax.ShapeDtypeStruct((N, D), jnp.float32),
    )(nf, proj_W, _row2d(proj_b), node_embeddings, degp3)

    s_p = _scatter_call(row3, col3, ew, y)

    w2p = jnp.zeros((D, D), jnp.float32).at[:, :OUT].set(pred_W2)
    b2p = jnp.zeros((1, D), jnp.float32).at[0, :OUT].set(pred_b2)

    logits_pad = pl.pallas_call(
        _dense_body,
        grid=grid,
        in_specs=[
            pl.BlockSpec((RB, 4), lambda i: (i, 0)),
            smallspec((3, D)),
            smallspec((1, D)),
            rowspec(),
            pl.BlockSpec((NC, RB, 1), lambda i: (0, i, 0)),
            pl.BlockSpec((NC, RB, D), lambda i: (0, i, 0)),
            smallspec((D, D)),
            smallspec((1, D)),
            smallspec((D, D)),
            smallspec((1, D)),
            smallspec((D, D)),
            smallspec((1, D)),
            smallspec((D, D)),
            smallspec((1, D)),
            smallspec((D, D)),
            smallspec((1, D)),
            smallspec((D, D)),
            smallspec((1, D)),
        ],
        out_specs=rowspec(),
        out_shape=jax.ShapeDtypeStruct((N, D), jnp.float32),
    )(nf, proj_W, _row2d(proj_b), node_embeddings, degp3, s_p,
      conv_z_W, _row2d(conv_z_b), conv_h_W, _row2d(conv_h_b),
      lin_z_W[:D], _row2d(lin_z_b), lin_h_W[:D], _row2d(lin_h_b),
      pred_W1, _row2d(pred_b1), w2p, b2p)

    return logits_pad[:, :OUT]


# speedup vs baseline: 1.0614x; 1.0614x over previous
import functools
import jax
import jax.numpy as jnp
from jax import lax
from jax.experimental import pallas as pl
from jax.experimental.pallas import tpu as pltpu
from jax.experimental.pallas import tpu_sc as plsc

N = 10000
D = 128
E = 320000
OUT = 3
NC = 2
NS = 16
NW = NC * NS
L = 16
EPT = E // NW
CH = 80
NCH = EPT // CH
DEG_PAD = 10240
DSL = DEG_PAD // NS
SPAD = 10240
RPT = SPAD // NS
RSTG = 128
SB = 25
NSB = NCH // SB
RB = 1000


def _zero_f32(ref, n):
    z = jnp.zeros((L,), jnp.float32)

    def body(i, c):
        ref[pl.ds(i * L, L)] = z
        return c

    lax.fori_loop(0, n // L, body, 0)


def _zero_rows(ref, rows):
    z = jnp.zeros((L,), jnp.float32)

    def body(i, c):
        def inner(d, c2):
            ref[i, pl.ds(d * L, L)] = z
            return c2

        return lax.fori_loop(0, D // L, inner, c)

    lax.fori_loop(0, rows, body, 0)


_SC_MESH = plsc.VectorSubcoreMesh(core_axis_name="c", subcore_axis_name="s")


@functools.partial(
    pl.kernel,
    mesh=_SC_MESH,
    out_type=jax.ShapeDtypeStruct((NC, DEG_PAD), jnp.float32),
    scratch_types=[
        pltpu.VMEM((NCH, 1, CH), jnp.int32),
        pltpu.VMEM((EPT,), jnp.float32),
        pltpu.VMEM((DSL,), jnp.float32),
        pltpu.VMEM_SHARED((DEG_PAD,), jnp.float32),
    ],
)
def _deg_call(col3_hbm, ew_hbm, out_hbm, cidx_v, ew_v, stage_v, deg_sh):
    cid = lax.axis_index("c")
    sid = lax.axis_index("s")
    wid = sid * NC + cid
    _zero_f32(stage_v, DSL)
    pltpu.sync_copy(stage_v, deg_sh.at[pl.ds(sid * DSL, DSL)])
    pltpu.sync_copy(col3_hbm.at[pl.ds(wid * NCH, NCH)], cidx_v)
    pltpu.sync_copy(ew_hbm.at[pl.ds(wid * EPT, EPT)], ew_v)
    plsc.subcore_barrier()

    def chunk(c, carry):
        pltpu.sync_copy(ew_v.at[pl.ds(c * CH, CH)],
                        deg_sh.at[cidx_v.at[c, 0]], add=True)
        return carry

    lax.fori_loop(0, NCH, chunk, 0)
    plsc.subcore_barrier()
    pltpu.sync_copy(deg_sh.at[pl.ds(sid * DSL, DSL)], stage_v)
    pltpu.sync_copy(stage_v, out_hbm.at[cid, pl.ds(sid * DSL, DSL)])


def _bcast_lane(v16, j):
    return lax.gather(
        v16, jnp.full((L, 1), j, jnp.int32),
        lax.GatherDimensionNumbers(
            offset_dims=(), collapsed_slice_dims=(0,), start_index_map=(0,)),
        slice_sizes=(1,),
        mode=lax.GatherScatterMode.PROMISE_IN_BOUNDS)


@functools.partial(
    pl.kernel,
    mesh=_SC_MESH,
    out_type=jax.ShapeDtypeStruct((NC, SPAD, D), jnp.float32),
    scratch_types=[
        pltpu.VMEM((SB, 1, CH), jnp.int32),
        pltpu.VMEM((SB, 1, CH), jnp.int32),
        pltpu.VMEM((SB * CH,), jnp.float32),
        pltpu.VMEM((CH, D), jnp.float32),
        pltpu.VMEM((CH, D), jnp.float32),
        pltpu.VMEM_SHARED((SPAD, D), jnp.float32),
        pltpu.SemaphoreType.DMA,
        pltpu.SemaphoreType.DMA,
        pltpu.SemaphoreType.DMA,
        pltpu.SemaphoreType.DMA,
    ],
)
def _scatter_call(row3_hbm, col3_hbm, ew_hbm, y_hbm, out_hbm,
                  ridx_v, cidx_v, ew_v, rows_a, rows_b, s_sh,
                  sem_a, sem_b, ssem_a, ssem_b):
    cid = lax.axis_index("c")
    sid = lax.axis_index("s")
    wid = sid * NC + cid
    _zero_rows(rows_a, CH)
    for k in range(RPT // CH):
        pltpu.sync_copy(rows_a, s_sh.at[pl.ds(sid * RPT + k * CH, CH)])
    plsc.subcore_barrier()

    def gather(c, buf, sem):
        pltpu.async_copy(y_hbm.at[ridx_v.at[c, 0]], buf, sem)

    def wait_g(c, buf, sem):
        pltpu.make_async_copy(y_hbm.at[ridx_v.at[c, 0]], buf, sem).wait()

    def scale(c, buf):
        @plsc.parallel_loop(0, CH // L, 1)
        def group(g):
            ew16 = ew_v[pl.ds(c * CH + g * L, L)]
            for j in range(L):
                w16 = _bcast_lane(ew16, j)
                e = g * L + j
                for d in range(D // L):
                    sl = pl.ds(d * L, L)
                    buf[e, sl] = buf[e, sl] * w16

    def issue_sc(c, buf, ssem):
        pltpu.async_copy(buf, s_sh.at[cidx_v.at[c, 0]], ssem, add=True)

    def wait_sc(c, buf, ssem):
        pltpu.make_async_copy(buf, s_sh.at[cidx_v.at[c, 0]], ssem).wait()

    def superblock(s, carry):
        cb = wid * NCH + s * SB
        pltpu.sync_copy(row3_hbm.at[pl.ds(cb, SB)], ridx_v)
        pltpu.sync_copy(col3_hbm.at[pl.ds(cb, SB)], cidx_v)
        pltpu.sync_copy(ew_hbm.at[pl.ds(wid * EPT + s * SB * CH, SB * CH)],
                        ew_v)
        gather(0, rows_a, sem_a)

        def pair(i, c2):
            c0 = 2 * i
            gather(c0 + 1, rows_b, sem_b)
            wait_g(c0, rows_a, sem_a)
            scale(c0, rows_a)
            issue_sc(c0, rows_a, ssem_a)
            wait_sc(c0, rows_a, ssem_a)
            gather(c0 + 2, rows_a, sem_a)
            wait_g(c0 + 1, rows_b, sem_b)
            scale(c0 + 1, rows_b)
            issue_sc(c0 + 1, rows_b, ssem_b)
            wait_sc(c0 + 1, rows_b, ssem_b)
            return c2

        lax.fori_loop(0, (SB - 1) // 2, pair, 0)
        wait_g(SB - 1, rows_a, sem_a)
        scale(SB - 1, rows_a)
        issue_sc(SB - 1, rows_a, ssem_a)
        wait_sc(SB - 1, rows_a, ssem_a)
        return carry

    lax.fori_loop(0, NSB, superblock, 0)
    plsc.subcore_barrier()
    for k in range(RPT // CH):
        r0 = sid * RPT + k * CH
        pltpu.sync_copy(s_sh.at[pl.ds(r0, CH)], rows_a)
        pltpu.sync_copy(rows_a, out_hbm.at[cid, pl.ds(r0, CH)])


def _y_body(nf_ref, pw_ref, pb_ref, emb_ref, degp_ref, y_ref):
    nf = nf_ref[...]
    x = (nf[:, 0:1] * pw_ref[0:1, :]
         + nf[:, 1:2] * pw_ref[1:2, :]
         + nf[:, 2:3] * pw_ref[2:3, :]
         + pb_ref[...] + emb_ref[...])
    deg = degp_ref[0] + degp_ref[1] + 1.0
    dis = lax.rsqrt(deg)
    y_ref[...] = dis * x


def _dense_body(nf_ref, pw_ref, pb_ref, emb_ref, degp_ref, sp_ref,
                wz_ref, bz_ref, wh_ref, bh_ref,
                lzw_ref, lzb_ref, lhw_ref, lhb_ref,
                w1_ref, b1_ref, w2_ref, b2_ref, out_ref):
    nf = nf_ref[...]
    x = (nf[:, 0:1] * pw_ref[0:1, :]
         + nf[:, 1:2] * pw_ref[1:2, :]
         + nf[:, 2:3] * pw_ref[2:3, :]
         + pb_ref[...] + emb_ref[...])
    deg = degp_ref[0] + degp_ref[1] + 1.0
    dis = lax.rsqrt(deg)
    s = sp_ref[0] + sp_ref[1]
    m = dis * s + (dis * dis) * x
    cz = jnp.dot(m, wz_ref[...], preferred_element_type=jnp.float32) + bz_ref[...]
    z = jax.nn.sigmoid(
        jnp.dot(cz, lzw_ref[...], preferred_element_type=jnp.float32) + lzb_ref[...])
    chh = jnp.dot(m, wh_ref[...], preferred_element_type=jnp.float32) + bh_ref[...]
    ht = jnp.tanh(
        jnp.dot(chh, lhw_ref[...], preferred_element_type=jnp.float32) + lhb_ref[...])
    hn = (1.0 - z) * ht
    hm = jnp.where(nf[:, 3:4] != -999.0, hn, 0.0)
    h1 = jax.nn.relu(
        jnp.dot(hm, w1_ref[...], preferred_element_type=jnp.float32) + b1_ref[...])
    out_ref[...] = (
        jnp.dot(h1, w2_ref[...], preferred_element_type=jnp.float32) + b2_ref[...])


def _row2d(v):
    return v.reshape(1, -1)


def kernel(dynamic_node_feats, node_ids, edge_index, edge_feats, proj_W,
           proj_b, node_embeddings, conv_z_W, conv_z_b, conv_r_W, conv_r_b,
           conv_h_W, conv_h_b, lin_z_W, lin_z_b, lin_r_W, lin_r_b, lin_h_W,
           lin_h_b, pred_W1, pred_b1, pred_W2, pred_b2):
    row = edge_index[0]
    col = edge_index[1]
    ew = edge_feats[:, 0]
    row3 = row.reshape(NW * NCH, 1, CH)
    col3 = col.reshape(NW * NCH, 1, CH)

    deg_p = _deg_call(col3, ew)
    degp3 = deg_p.reshape(NC, DEG_PAD, 1)

    nf = dynamic_node_feats
    grid = (N // RB,)

    def rowspec():
        return pl.BlockSpec((RB, D), lambda i: (i, 0))

    def smallspec(shape):
        return pl.BlockSpec(shape, lambda i: tuple(0 for _ in shape))

    y = pl.pallas_call(
        _y_body,
        grid=grid,
        in_specs=[
            pl.BlockSpec((RB, 4), lambda i: (i, 0)),
            smallspec((3, D)),
            smallspec((1, D)),
            rowspec(),
            pl.BlockSpec((NC, RB, 1), lambda i: (0, i, 0)),
        ],
        out_specs=rowspec(),
        out_shape=jax.ShapeDtypeStruct((N, D), jnp.float32),
    )(nf, proj_W, _row2d(proj_b), node_embeddings, degp3)

    s_p = _scatter_call(row3, col3, ew, y)

    w2p = jnp.zeros((D, D), jnp.float32).at[:, :OUT].set(pred_W2)
    b2p = jnp.zeros((1, D), jnp.float32).at[0, :OUT].set(pred_b2)

    logits_pad = pl.pallas_call(
        _dense_body,
        grid=grid,
        in_specs=[
            pl.BlockSpec((RB, 4), lambda i: (i, 0)),
            smallspec((3, D)),
            smallspec((1, D)),
            rowspec(),
            pl.BlockSpec((NC, RB, 1), lambda i: (0, i, 0)),
            pl.BlockSpec((NC, RB, D), lambda i: (0, i, 0)),
            smallspec((D, D)),
            smallspec((1, D)),
            smallspec((D, D)),
            smallspec((1, D)),
            smallspec((D, D)),
            smallspec((1, D)),
            smallspec((D, D)),
            smallspec((1, D)),
            smallspec((D, D)),
            smallspec((1, D)),
            smallspec((D, D)),
            smallspec((1, D)),
        ],
        out_specs=rowspec(),
        out_shape=jax.ShapeDtypeStruct((N, D), jnp.float32),
    )(nf, proj_W, _row2d(proj_b), node_embeddings, degp3, s_p,
      conv_z_W, _row2d(conv_z_b), conv_h_W, _row2d(conv_h_b),
      lin_z_W[:D], _row2d(lin_z_b), lin_h_W[:D], _row2d(lin_h_b),
      pred_W1, _row2d(pred_b1), w2p, b2p)

    return logits_pad[:, :OUT]


# speedup vs baseline: 1.1199x; 1.0551x over previous
import functools
import jax
import jax.numpy as jnp
from jax import lax
from jax.experimental import pallas as pl
from jax.experimental.pallas import tpu as pltpu
from jax.experimental.pallas import tpu_sc as plsc

N = 10000
D = 128
E = 320000
OUT = 3
NC = 2
NS = 16
NW = NC * NS
L = 16
EPT = E // NW
CH = 80
NCH = EPT // CH
DEG_PAD = 10240
DSL = DEG_PAD // NS
SPAD = 10240
RPT = SPAD // NS
RSTG = 128
SB = 25
NSB = NCH // SB
RB = 1000


def _zero_f32(ref, n):
    z = jnp.zeros((L,), jnp.float32)

    def body(i, c):
        ref[pl.ds(i * L, L)] = z
        return c

    lax.fori_loop(0, n // L, body, 0)


def _zero_rows(ref, rows):
    z = jnp.zeros((L,), jnp.float32)

    def body(i, c):
        def inner(d, c2):
            ref[i, pl.ds(d * L, L)] = z
            return c2

        return lax.fori_loop(0, D // L, inner, c)

    lax.fori_loop(0, rows, body, 0)


_SC_MESH = plsc.VectorSubcoreMesh(core_axis_name="c", subcore_axis_name="s")


@functools.partial(
    pl.kernel,
    mesh=_SC_MESH,
    out_type=jax.ShapeDtypeStruct((NC, DEG_PAD), jnp.float32),
    scratch_types=[
        pltpu.VMEM((NCH, 1, CH), jnp.int32),
        pltpu.VMEM((EPT,), jnp.float32),
        pltpu.VMEM((DSL,), jnp.float32),
        pltpu.VMEM_SHARED((DEG_PAD,), jnp.float32),
    ],
)
def _deg_call(col3_hbm, ew_hbm, out_hbm, cidx_v, ew_v, stage_v, deg_sh):
    cid = lax.axis_index("c")
    sid = lax.axis_index("s")
    wid = sid * NC + cid
    _zero_f32(stage_v, DSL)
    pltpu.sync_copy(stage_v, deg_sh.at[pl.ds(sid * DSL, DSL)])
    pltpu.sync_copy(col3_hbm.at[pl.ds(wid * NCH, NCH)], cidx_v)
    pltpu.sync_copy(ew_hbm.at[pl.ds(wid * EPT, EPT)], ew_v)
    plsc.subcore_barrier()

    def chunk(c, carry):
        pltpu.sync_copy(ew_v.at[pl.ds(c * CH, CH)],
                        deg_sh.at[cidx_v.at[c, 0]], add=True)
        return carry

    lax.fori_loop(0, NCH, chunk, 0)
    plsc.subcore_barrier()
    pltpu.sync_copy(deg_sh.at[pl.ds(sid * DSL, DSL)], stage_v)
    pltpu.sync_copy(stage_v, out_hbm.at[cid, pl.ds(sid * DSL, DSL)])


def _rsqrt16(d):
    i = lax.bitcast_convert_type(d, jnp.int32)
    i = jnp.int32(0x5F3759DF) - lax.shift_right_logical(i, 1)
    r = lax.bitcast_convert_type(i, jnp.float32)
    for _ in range(3):
        r = r * (1.5 - 0.5 * d * r * r)
    return r


def _bcast_lane(v16, j):
    return lax.gather(
        v16, jnp.full((L, 1), j, jnp.int32),
        lax.GatherDimensionNumbers(
            offset_dims=(), collapsed_slice_dims=(0,), start_index_map=(0,)),
        slice_sizes=(1,),
        mode=lax.GatherScatterMode.PROMISE_IN_BOUNDS)


@functools.partial(
    pl.kernel,
    mesh=_SC_MESH,
    out_type=jax.ShapeDtypeStruct((NC, SPAD, D), jnp.float32),
    scratch_types=[
        pltpu.VMEM((SB, 1, CH), jnp.int32),
        pltpu.VMEM((SB, 1, CH), jnp.int32),
        pltpu.VMEM((SB * CH,), jnp.float32),
        pltpu.VMEM((CH, D), jnp.float32),
        pltpu.VMEM((CH, D), jnp.float32),
        pltpu.VMEM((CH,), jnp.float32),
        pltpu.VMEM((CH,), jnp.float32),
        pltpu.VMEM((DSL,), jnp.float32),
        pltpu.VMEM((DSL,), jnp.float32),
        pltpu.VMEM_SHARED((SPAD, D), jnp.float32),
        pltpu.VMEM_SHARED((DEG_PAD,), jnp.float32),
        pltpu.SemaphoreType.DMA,
        pltpu.SemaphoreType.DMA,
        pltpu.SemaphoreType.DMA,
        pltpu.SemaphoreType.DMA,
        pltpu.SemaphoreType.DMA,
        pltpu.SemaphoreType.DMA,
    ],
)
def _scatter_call(row3_hbm, col3_hbm, ew_hbm, x_hbm, degp_hbm, out_hbm,
                  ridx_v, cidx_v, ew_v, rows_a, rows_b, dis_a, dis_b,
                  p0_v, p1_v, s_sh, dis_sh,
                  sem_a, sem_b, ssem_a, ssem_b, dsem_a, dsem_b):
    cid = lax.axis_index("c")
    sid = lax.axis_index("s")
    wid = sid * NC + cid
    pltpu.sync_copy(degp_hbm.at[0, pl.ds(sid * DSL, DSL)], p0_v)
    pltpu.sync_copy(degp_hbm.at[1, pl.ds(sid * DSL, DSL)], p1_v)

    def mkdis(i, carry):
        sl = pl.ds(i * L, L)
        p0_v[sl] = _rsqrt16(p0_v[sl] + p1_v[sl] + 1.0)
        return carry

    lax.fori_loop(0, DSL // L, mkdis, 0)
    pltpu.sync_copy(p0_v, dis_sh.at[pl.ds(sid * DSL, DSL)])
    _zero_rows(rows_a, CH)
    for k in range(RPT // CH):
        pltpu.sync_copy(rows_a, s_sh.at[pl.ds(sid * RPT + k * CH, CH)])
    plsc.subcore_barrier()

    def gather(c, buf, sem, dbuf, dsem):
        pltpu.async_copy(x_hbm.at[ridx_v.at[c, 0]], buf, sem)
        pltpu.async_copy(dis_sh.at[ridx_v.at[c, 0]], dbuf, dsem)

    def wait_g(c, buf, sem, dbuf, dsem):
        pltpu.make_async_copy(x_hbm.at[ridx_v.at[c, 0]], buf, sem).wait()
        pltpu.make_async_copy(dis_sh.at[ridx_v.at[c, 0]], dbuf, dsem).wait()

    def scale(c, buf, dbuf):
        @plsc.parallel_loop(0, CH // L, 1)
        def group(g):
            ew16 = ew_v[pl.ds(c * CH + g * L, L)] * dbuf[pl.ds(g * L, L)]
            for j in range(L):
                w16 = _bcast_lane(ew16, j)
                e = g * L + j
                for d in range(D // L):
                    sl = pl.ds(d * L, L)
                    buf[e, sl] = buf[e, sl] * w16

    def issue_sc(c, buf, ssem):
        pltpu.async_copy(buf, s_sh.at[cidx_v.at[c, 0]], ssem, add=True)

    def wait_sc(c, buf, ssem):
        pltpu.make_async_copy(buf, s_sh.at[cidx_v.at[c, 0]], ssem).wait()

    def superblock(s, carry):
        cb = wid * NCH + s * SB
        pltpu.sync_copy(row3_hbm.at[pl.ds(cb, SB)], ridx_v)
        pltpu.sync_copy(col3_hbm.at[pl.ds(cb, SB)], cidx_v)
        pltpu.sync_copy(ew_hbm.at[pl.ds(wid * EPT + s * SB * CH, SB * CH)],
                        ew_v)
        gather(0, rows_a, sem_a, dis_a, dsem_a)

        def pair(i, c2):
            c0 = 2 * i
            gather(c0 + 1, rows_b, sem_b, dis_b, dsem_b)
            wait_g(c0, rows_a, sem_a, dis_a, dsem_a)
            scale(c0, rows_a, dis_a)
            issue_sc(c0, rows_a, ssem_a)
            wait_sc(c0, rows_a, ssem_a)
            gather(c0 + 2, rows_a, sem_a, dis_a, dsem_a)
            wait_g(c0 + 1, rows_b, sem_b, dis_b, dsem_b)
            scale(c0 + 1, rows_b, dis_b)
            issue_sc(c0 + 1, rows_b, ssem_b)
            wait_sc(c0 + 1, rows_b, ssem_b)
            return c2

        lax.fori_loop(0, (SB - 1) // 2, pair, 0)
        wait_g(SB - 1, rows_a, sem_a, dis_a, dsem_a)
        scale(SB - 1, rows_a, dis_a)
        issue_sc(SB - 1, rows_a, ssem_a)
        wait_sc(SB - 1, rows_a, ssem_a)
        return carry

    lax.fori_loop(0, NSB, superblock, 0)
    plsc.subcore_barrier()
    for k in range(RPT // CH):
        r0 = sid * RPT + k * CH
        pltpu.sync_copy(s_sh.at[pl.ds(r0, CH)], rows_a)
        pltpu.sync_copy(rows_a, out_hbm.at[cid, pl.ds(r0, CH)])


def _x_body(nf_ref, pw_ref, pb_ref, emb_ref, x_ref):
    nf = nf_ref[...]
    x_ref[...] = (nf[:, 0:1] * pw_ref[0:1, :]
                  + nf[:, 1:2] * pw_ref[1:2, :]
                  + nf[:, 2:3] * pw_ref[2:3, :]
                  + pb_ref[...] + emb_ref[...])


def _dense_body(nf_ref, pw_ref, pb_ref, emb_ref, degp_ref, sp_ref,
                wz_ref, bz_ref, wh_ref, bh_ref,
                lzw_ref, lzb_ref, lhw_ref, lhb_ref,
                w1_ref, b1_ref, w2_ref, b2_ref, out_ref):
    nf = nf_ref[...]
    x = (nf[:, 0:1] * pw_ref[0:1, :]
         + nf[:, 1:2] * pw_ref[1:2, :]
         + nf[:, 2:3] * pw_ref[2:3, :]
         + pb_ref[...] + emb_ref[...])
    deg = degp_ref[0] + degp_ref[1] + 1.0
    dis = lax.rsqrt(deg)
    s = sp_ref[0] + sp_ref[1]
    m = dis * s + (dis * dis) * x
    cz = jnp.dot(m, wz_ref[...], preferred_element_type=jnp.float32) + bz_ref[...]
    z = jax.nn.sigmoid(
        jnp.dot(cz, lzw_ref[...], preferred_element_type=jnp.float32) + lzb_ref[...])
    chh = jnp.dot(m, wh_ref[...], preferred_element_type=jnp.float32) + bh_ref[...]
    ht = jnp.tanh(
        jnp.dot(chh, lhw_ref[...], preferred_element_type=jnp.float32) + lhb_ref[...])
    hn = (1.0 - z) * ht
    hm = jnp.where(nf[:, 3:4] != -999.0, hn, 0.0)
    h1 = jax.nn.relu(
        jnp.dot(hm, w1_ref[...], preferred_element_type=jnp.float32) + b1_ref[...])
    out_ref[...] = (
        jnp.dot(h1, w2_ref[...], preferred_element_type=jnp.float32) + b2_ref[...])


def _row2d(v):
    return v.reshape(1, -1)


def kernel(dynamic_node_feats, node_ids, edge_index, edge_feats, proj_W,
           proj_b, node_embeddings, conv_z_W, conv_z_b, conv_r_W, conv_r_b,
           conv_h_W, conv_h_b, lin_z_W, lin_z_b, lin_r_W, lin_r_b, lin_h_W,
           lin_h_b, pred_W1, pred_b1, pred_W2, pred_b2):
    row = edge_index[0]
    col = edge_index[1]
    ew = edge_feats[:, 0]
    row3 = row.reshape(NW * NCH, 1, CH)
    col3 = col.reshape(NW * NCH, 1, CH)

    deg_p = _deg_call(col3, ew)
    degp3 = deg_p.reshape(NC, DEG_PAD, 1)

    nf = dynamic_node_feats
    grid = (N // RB,)

    def rowspec():
        return pl.BlockSpec((RB, D), lambda i: (i, 0))

    def smallspec(shape):
        return pl.BlockSpec(shape, lambda i: tuple(0 for _ in shape))

    x = pl.pallas_call(
        _x_body,
        grid=grid,
        in_specs=[
            pl.BlockSpec((RB, 4), lambda i: (i, 0)),
            smallspec((3, D)),
            smallspec((1, D)),
            rowspec(),
        ],
        out_specs=rowspec(),
        out_shape=jax.ShapeDtypeStruct((N, D), jnp.float32),
    )(nf, proj_W, _row2d(proj_b), node_embeddings)

    s_p = _scatter_call(row3, col3, ew, x, deg_p)

    w2p = jnp.zeros((D, D), jnp.float32).at[:, :OUT].set(pred_W2)
    b2p = jnp.zeros((1, D), jnp.float32).at[0, :OUT].set(pred_b2)

    logits_pad = pl.pallas_call(
        _dense_body,
        grid=grid,
        in_specs=[
            pl.BlockSpec((RB, 4), lambda i: (i, 0)),
            smallspec((3, D)),
            smallspec((1, D)),
            rowspec(),
            pl.BlockSpec((NC, RB, 1), lambda i: (0, i, 0)),
            pl.BlockSpec((NC, RB, D), lambda i: (0, i, 0)),
            smallspec((D, D)),
            smallspec((1, D)),
            smallspec((D, D)),
            smallspec((1, D)),
            smallspec((D, D)),
            smallspec((1, D)),
            smallspec((D, D)),
            smallspec((1, D)),
            smallspec((D, D)),
            smallspec((1, D)),
            smallspec((D, D)),
            smallspec((1, D)),
        ],
        out_specs=rowspec(),
        out_shape=jax.ShapeDtypeStruct((N, D), jnp.float32),
    )(nf, proj_W, _row2d(proj_b), node_embeddings, degp3, s_p,
      conv_z_W, _row2d(conv_z_b), conv_h_W, _row2d(conv_h_b),
      lin_z_W[:D], _row2d(lin_z_b), lin_h_W[:D], _row2d(lin_h_b),
      pred_W1, _row2d(pred_b1), w2p, b2p)

    return logits_pad[:, :OUT]
